# Initial kernel scaffold; baseline (speedup 1.0000x reference)
import jax
import jax.numpy as jnp
from jax.experimental import pallas as pl


def kernel(x_num_t, eps_pred, pred, noise, uniform, x_cat_idx, t, beta):
    raise NotImplementedError("write your pallas kernel here")


# speedup vs baseline: 2.4807x; 2.4807x over previous
import functools

import numpy as np
import jax
import jax.numpy as jnp
from jax import lax
from jax.experimental import pallas as pl
from jax.experimental.pallas import tpu as pltpu
from jax.experimental.pallas import tpu_sc as plsc

NUM_FIELDS = 26
CLASSES = 32
CAT_DIM = NUM_FIELDS * CLASSES
NUM_NUM = 16
T_STEPS = 1000
T_PAD = 1024
NCOEF = 128
BATCH = 16384
BLK = 512
LOGK = float(np.log(np.float32(CLASSES)))
NEG = float(np.log(np.float32(1e-30)))

SC_CORES = 2
SC_SUBCORES = 16
NW = SC_CORES * SC_SUBCORES
BPW = BATCH // NW
CHUNK = 128


def _schedule_body(beta_ref, tab_ref):
    beta = beta_ref[...]
    alpha = 1.0 - beta
    la = jnp.log(alpha)
    row = lax.broadcasted_iota(jnp.int32, (T_PAD, T_PAD), 0)
    col = lax.broadcasted_iota(jnp.int32, (T_PAD, T_PAD), 1)
    tri = (col <= row).astype(jnp.float32)
    s = jnp.dot(tri, la, preferred_element_type=jnp.float32)
    s_prev = s - la
    ca = jnp.exp(s)
    one_min = 1.0 - ca
    ca_prev = jnp.exp(s_prev)
    one_min_prev = 1.0 - ca_prev
    sqrt_ca = jnp.sqrt(ca)
    sqrt_1m = jnp.sqrt(one_min)
    coef1 = beta * jnp.sqrt(ca_prev) / one_min
    coef2 = one_min_prev * jnp.sqrt(alpha) / one_min
    sqrt_pv = jnp.sqrt(beta * one_min_prev / one_min)
    log_1m_ca_prev = jnp.maximum(jnp.log(one_min_prev), -1e30)
    lb = jnp.log(beta)
    zeros = jnp.zeros((T_PAD, NCOEF - 9), jnp.float32)
    tab_ref[...] = jnp.concatenate(
        [sqrt_ca, sqrt_1m, coef1, coef2, sqrt_pv, la, lb, s_prev,
         log_1m_ca_prev, zeros],
        axis=1)


def _gather_coeffs(table, t):
    mesh = plsc.VectorSubcoreMesh(core_axis_name="c", subcore_axis_name="s")

    @functools.partial(
        pl.kernel,
        mesh=mesh,
        out_type=jax.ShapeDtypeStruct((BATCH, NCOEF), jnp.float32),
        scratch_types=[
            pltpu.VMEM((BPW,), jnp.int32),
            pltpu.VMEM((BPW, NCOEF), jnp.float32),
            pltpu.SemaphoreType.DMA,
        ],
    )
    def k(tab_hbm, t_hbm, out_hbm, idx_v, rows_v, sem):
        wid = lax.axis_index("s") * SC_CORES + lax.axis_index("c")
        base = wid * BPW
        pltpu.sync_copy(t_hbm.at[pl.ds(base, BPW)], idx_v)
        copies = [
            pltpu.async_copy(
                tab_hbm.at[idx_v.at[pl.ds(j * CHUNK, CHUNK)]],
                rows_v.at[pl.ds(j * CHUNK, CHUNK)],
                sem,
            )
            for j in range(BPW // CHUNK)
        ]
        for cp in copies:
            cp.wait()
        pltpu.sync_copy(rows_v, out_hbm.at[pl.ds(base, BPW)])

    return k(table, t)


def _rot(x, k):
    return jnp.concatenate([x[:, k:], x[:, :k]], axis=1)


def _seg_max(x):
    pos = lax.broadcasted_iota(jnp.int32, x.shape, 1) % CLASSES
    n = x.shape[1]
    for k in (1, 2, 4, 8, 16):
        r1 = _rot(x, k)
        r2 = _rot(x, n - (CLASSES - k))
        x = jnp.maximum(x, jnp.where(pos < CLASSES - k, r1, r2))
    return x


def _main_body(xn_ref, ep_ref, nz_ref, pred_ref, u_ref, idx_ref, cf_ref,
               out_ref, ls_ref):
    blk = xn_ref.shape[0]
    cf = cf_ref[...]
    c0 = cf[:, 0:1]
    c1 = cf[:, 1:2]
    c2 = cf[:, 2:3]
    c3 = cf[:, 3:4]
    c4 = cf[:, 4:5]
    c5 = cf[:, 5:6]
    c6 = cf[:, 6:7]
    c7 = cf[:, 7:8]
    c8 = cf[:, 8:9]

    x = xn_ref[...]
    x0 = (x - c1 * ep_ref[...]) / c0
    gauss = c2 * x0 + c3 * x + c4 * nz_ref[...]

    erow = lax.broadcasted_iota(jnp.int32, (NUM_FIELDS, CAT_DIM), 0)
    ecol = lax.broadcasted_iota(jnp.int32, (NUM_FIELDS, CAT_DIM), 1) // CLASSES
    E = (erow == ecol).astype(jnp.float32)
    srow = lax.broadcasted_iota(jnp.int32, (CAT_DIM, NUM_FIELDS), 0) // CLASSES
    scol = lax.broadcasted_iota(jnp.int32, (CAT_DIM, NUM_FIELDS), 1)
    S = (srow == scol).astype(jnp.float32)

    pred = pred_ref[...]
    e1 = jnp.exp(pred)
    lse1 = jnp.log(jnp.dot(e1, S, preferred_element_type=jnp.float32))
    log_x0 = pred - jnp.dot(lse1, E, preferred_element_type=jnp.float32)

    a = log_x0 + c7
    b = c8 - LOGK
    m = jnp.maximum(a, b)
    log_ev = m + jnp.log(jnp.exp(a - m) + jnp.exp(b - m))

    idxf = idx_ref[...].astype(jnp.float32)
    idxb = jnp.dot(idxf, E, preferred_element_type=jnp.float32)
    pos = lax.broadcasted_iota(jnp.int32, (blk, CAT_DIM), 1) % CLASSES
    onehot = jnp.abs(idxb - pos.astype(jnp.float32)) < 0.5
    b2 = c6 - LOGK
    mq = jnp.maximum(c5, b2)
    qa = mq + jnp.log(jnp.exp(c5 - mq) + jnp.exp(b2 - mq))
    q_one = jnp.where(onehot, qa, b2)

    z = log_ev + q_one
    s2 = jnp.dot(jnp.exp(z), S, preferred_element_type=jnp.float32)
    log_dist = z - jnp.dot(jnp.log(s2), E, preferred_element_type=jnp.float32)

    g = -jnp.log(1e-30 - jnp.log(u_ref[...] + 1e-30))
    pert = g + log_dist
    mx = _seg_max(pert)
    ls = jnp.where(pert == mx, 0.0, NEG)

    out_ref[...] = jnp.concatenate([gauss, log_dist], axis=1)
    ls_ref[...] = ls


def _main_call(x_num_t, eps_pred, noise, pred, uniform, x_cat_idx, coeffs,
               interpret=False):
    nblk = BATCH // BLK
    return pl.pallas_call(
        _main_body,
        grid=(nblk,),
        in_specs=[
            pl.BlockSpec((BLK, NUM_NUM), lambda i: (i, 0)),
            pl.BlockSpec((BLK, NUM_NUM), lambda i: (i, 0)),
            pl.BlockSpec((BLK, NUM_NUM), lambda i: (i, 0)),
            pl.BlockSpec((BLK, CAT_DIM), lambda i: (i, 0)),
            pl.BlockSpec((BLK, CAT_DIM), lambda i: (i, 0)),
            pl.BlockSpec((BLK, NUM_FIELDS), lambda i: (i, 0)),
            pl.BlockSpec((BLK, NCOEF), lambda i: (i, 0)),
        ],
        out_specs=[
            pl.BlockSpec((BLK, NUM_NUM + CAT_DIM), lambda i: (i, 0)),
            pl.BlockSpec((BLK, CAT_DIM), lambda i: (i, 0)),
        ],
        out_shape=[
            jax.ShapeDtypeStruct((BATCH, NUM_NUM + CAT_DIM), jnp.float32),
            jax.ShapeDtypeStruct((BATCH, CAT_DIM), jnp.float32),
        ],
        interpret=interpret,
    )(x_num_t, eps_pred, noise, pred, uniform, x_cat_idx, coeffs)


def _schedule_call(beta, interpret=False):
    beta_p = jnp.pad(beta.astype(jnp.float32), (0, T_PAD - T_STEPS),
                     constant_values=0.5).reshape(T_PAD, 1)
    return pl.pallas_call(
        _schedule_body,
        out_shape=jax.ShapeDtypeStruct((T_PAD, NCOEF), jnp.float32),
        interpret=interpret,
    )(beta_p)


def kernel(x_num_t, eps_pred, pred, noise, uniform, x_cat_idx, t, beta):
    table = _schedule_call(beta)
    coeffs = _gather_coeffs(table, t.astype(jnp.int32))
    out, ls = _main_call(x_num_t, eps_pred, noise, pred, uniform,
                         x_cat_idx, coeffs)
    return (out, ls)
